# baseline (speedup 1.0000x reference)
import functools

import jax
import jax.numpy as jnp
import numpy as np
from jax import lax
from jax.experimental import pallas as pl
from jax.experimental.pallas import tpu as pltpu
from jax.experimental.pallas import tpu_sc as plsc

B, N, M, D, C = 4, 8192, 1024, 3, 32
K = 32
R2 = np.float32(0.6 ** 2)
OP_CH, OUT_CH = 64, 128
XP = 40
CP = 8
BM = B * M
BN = B * N
NW = 32
RPW = BM // NW
GR = 8
NG = RPW // GR
GI = GR * K // 128
PW = 128
CH0 = 256
NS0 = CH0 // 16
CHF = 512
NSF = CHF // 16
NCF = N // CHF
NEG = np.float32(-3.0e38)


def _point_mlp_body(pos_ref, feat_ref, wp_ref, wf_ref, o_ref):
    p = (jnp.dot(pos_ref[...], wp_ref[...],
                 preferred_element_type=jnp.float32)
         + jnp.dot(feat_ref[...], wf_ref[...],
                   preferred_element_type=jnp.float32))
    o_ref[...] = jnp.concatenate(
        [p, jnp.zeros((p.shape[0], PW - OP_CH), jnp.float32)], axis=1)


def _head_body(q_ref, c_ref, wp_ref, bop_ref, wa_ref, ba_ref, o_ref):
    cq = jnp.dot(c_ref[...], wp_ref[...], preferred_element_type=jnp.float32)
    pooled = jnp.maximum(q_ref[:, :OP_CH] - cq + bop_ref[...], 0.0)
    o_ref[...] = jnp.maximum(
        jnp.dot(pooled, wa_ref[...], preferred_element_type=jnp.float32)
        + ba_ref[...], 0.0)


def _lane_gather(x, idx):
    return lax.gather(
        x, idx[:, None],
        dimension_numbers=lax.GatherDimensionNumbers(
            offset_dims=(), collapsed_slice_dims=(0,), start_index_map=(0,)),
        slice_sizes=(1,),
        mode=lax.GatherScatterMode.PROMISE_IN_BOUNDS)


def _sc_body(dist_hbm, p_hbm, q_hbm,
             dbuf, fbuf, ids, idsall, prows, qloc, cnts, cnt_ref,
             gsem, hsem):
    cid = lax.axis_index("c")
    sid = lax.axis_index("s")
    wid = sid * 2 + cid
    row0 = wid * RPW
    gbase = (row0 // M) * N
    iot = lax.iota(jnp.int32, 16)
    one = jnp.full((16,), 1, jnp.int32)
    zero = jnp.full((16,), 0, jnp.int32)
    negv = jnp.full((16,), NEG, jnp.float32)

    pidx = [jnp.maximum(iot - s, 0) for s in (1, 2, 4, 8)]
    pmsk = [iot >= s for s in (1, 2, 4, 8)]

    def subchunk(v, base, c):
        m = v < R2
        x = jnp.where(m, one, zero)
        for i in range(4):
            sh = _lane_gather(x, pidx[i])
            x = x + jnp.where(pmsk[i], sh, zero)
        pc = x[15]
        r = zero
        for s in (8, 4, 2, 1):
            t = r + s
            ok = _lane_gather(x, t - 1) <= iot
            r = jnp.where(ok, t, r)
        ids[pl.ds(jnp.minimum(c, 34), 16)] = r + base
        return c + pc

    def scan_row(r, grow0, par):
        cnt_ref[0] = jnp.int32(0)
        prow = par * GR + r

        def head(jq, carry):
            @pl.when(cnt_ref[0] < K)
            def _():
                c = cnt_ref[0]
                for j4 in range(4):
                    j = jq * 4 + j4
                    c = subchunk(dbuf[prow, pl.ds(j * 16, 16)],
                                 j * 16 + gbase, c)
                cnt_ref[0] = c
            return carry

        lax.fori_loop(0, NS0 // 4, head, 0)

        @pl.when(cnt_ref[0] < K)
        def _():
            def fb_chunk(f, carry):
                @pl.when(cnt_ref[0] < K)
                def _():
                    pltpu.sync_copy(
                        dist_hbm.at[pl.ds(grow0, GR), pl.ds(f * CHF, CHF)],
                        fbuf)

                    def fb_sub(j, c2):
                        @pl.when(cnt_ref[0] < K)
                        def _():
                            cnt_ref[0] = subchunk(
                                fbuf[r, pl.ds(j * 16, 16)],
                                f * CHF + j * 16 + gbase, cnt_ref[0])
                        return c2

                    lax.fori_loop(0, NSF, fb_sub, 0)
                return carry

            lax.fori_loop(1, NCF, fb_chunk, 0)

        cnt = cnt_ref[0]
        cntc = jnp.minimum(cnt, K)
        fill = _lane_gather(ids[pl.ds(0, 16)], zero)
        ids[pl.ds(cntc, 16)] = fill
        ids[pl.ds(cntc + 16, 16)] = fill
        rq = par * GI + r // 4
        rr = (r % 4) * K
        idsall[rq, pl.ds(rr, 16)] = ids[pl.ds(0, 16)]
        idsall[rq, pl.ds(rr + 16, 16)] = ids[pl.ds(16, 16)]
        cnts[prow] = cnt

    def head_copy(g, par):
        return pltpu.make_async_copy(
            dist_hbm.at[pl.ds(row0 + g * GR, GR), pl.ds(0, CH0)],
            dbuf.at[pl.ds(par * GR, GR)], hsem)

    def gather_copy(par, gi):
        return pltpu.make_async_copy(
            p_hbm.at[idsall.at[par * GI + gi]],
            prows.at[pl.ds(par * GR * K + gi * 128, 128)], gsem)

    def max_group(gq, par):
        def mrow(r, c2):
            accs = [negv] * 4
            for k in range(K):
                for t in range(4):
                    v = prows[par * GR * K + r * K + k, pl.ds(t * 16, 16)]
                    accs[t] = jnp.maximum(accs[t], v)
            valid = cnts[par * GR + r] > 0

            @pl.when(valid)
            def _():
                for t in range(4):
                    qloc[gq * GR + r, pl.ds(t * 16, 16)] = accs[t]

            @pl.when(jnp.logical_not(valid))
            def _():
                for t in range(4):
                    qloc[gq * GR + r, pl.ds(t * 16, 16)] = negv
            return c2

        lax.fori_loop(0, GR, mrow, 0)

    head_copy(0, 0).start()

    def group(g, carry):
        par = g & 1
        parp = 1 - par
        grow0 = row0 + g * GR
        head_copy(g, par).wait()

        @pl.when(g + 1 < NG)
        def _():
            head_copy(g + 1, parp).start()

        def srow(r, c2):
            scan_row(r, grow0, par)
            return c2

        lax.fori_loop(0, GR, srow, 0)

        @pl.when(g > 0)
        def _():
            for gi in range(GI):
                gather_copy(parp, gi).wait()
            max_group(g - 1, parp)

        for gi in range(GI):
            gather_copy(par, gi).start()
        return carry

    lax.fori_loop(0, NG, group, 0)
    parl = (NG - 1) & 1
    for gi in range(GI):
        gather_copy(parl, gi).wait()
    max_group(NG - 1, parl)
    pltpu.sync_copy(qloc, q_hbm.at[pl.ds(row0, RPW)])


_sc_ball = functools.partial(
    pl.kernel,
    out_type=jax.ShapeDtypeStruct((BM, PW), jnp.float32),
    mesh=plsc.VectorSubcoreMesh(core_axis_name="c", subcore_axis_name="s"),
    compiler_params=pltpu.CompilerParams(use_tc_tiling_on_sc=True),
    scratch_types=[
        pltpu.VMEM((2 * GR, CH0), jnp.float32),
        pltpu.VMEM((GR, CHF), jnp.float32),
        pltpu.VMEM((64,), jnp.int32),
        pltpu.VMEM((2 * GI, 128), jnp.int32),
        pltpu.VMEM((2 * GR * K, PW), jnp.float32),
        pltpu.VMEM((RPW, PW), jnp.float32),
        pltpu.SMEM((2 * GR,), jnp.int32),
        pltpu.SMEM((1,), jnp.int32),
        pltpu.SemaphoreType.DMA,
        pltpu.SemaphoreType.DMA,
    ],
)(_sc_body)


@jax.jit
def kernel(positions, features, centers, distances, W_op, b_op, W_agg, b_agg):
    c8 = jnp.pad(centers.reshape(BM, D), ((0, 0), (0, CP - D)))
    wp8 = jnp.pad(W_op[:D], ((0, CP - D), (0, 0)))

    p = pl.pallas_call(
        _point_mlp_body,
        grid=(BN // 16384,),
        in_specs=[pl.BlockSpec((16384, D), lambda i: (i, 0)),
                  pl.BlockSpec((16384, C), lambda i: (i, 0)),
                  pl.BlockSpec((D, OP_CH), lambda i: (0, 0)),
                  pl.BlockSpec((C, OP_CH), lambda i: (0, 0))],
        out_specs=pl.BlockSpec((16384, PW), lambda i: (i, 0)),
        out_shape=jax.ShapeDtypeStruct((BN, PW), jnp.float32),
    )(positions.reshape(BN, D), features.reshape(BN, C),
      W_op[:D], W_op[D:])

    q = _sc_ball(distances.reshape(BM, N), p)

    out = pl.pallas_call(
        _head_body,
        grid=(BM // 4096,),
        in_specs=[pl.BlockSpec((4096, PW), lambda i: (i, 0)),
                  pl.BlockSpec((4096, CP), lambda i: (i, 0)),
                  pl.BlockSpec((CP, OP_CH), lambda i: (0, 0)),
                  pl.BlockSpec((1, OP_CH), lambda i: (0, 0)),
                  pl.BlockSpec((OP_CH, OUT_CH), lambda i: (0, 0)),
                  pl.BlockSpec((1, OUT_CH), lambda i: (0, 0))],
        out_specs=pl.BlockSpec((4096, OUT_CH), lambda i: (i, 0)),
        out_shape=jax.ShapeDtypeStruct((BM, OUT_CH), jnp.float32),
    )(q, c8, wp8, b_op.reshape(1, OP_CH), W_agg, b_agg.reshape(1, OUT_CH))

    return out.reshape(B, M, OUT_CH)


# speedup vs baseline: 1.0136x; 1.0136x over previous
import functools

import jax
import jax.numpy as jnp
import numpy as np
from jax import lax
from jax.experimental import pallas as pl
from jax.experimental.pallas import tpu as pltpu
from jax.experimental.pallas import tpu_sc as plsc

B, N, M, D, C = 4, 8192, 1024, 3, 32
K = 32
R2 = np.float32(0.6 ** 2)
OP_CH, OUT_CH = 64, 128
XP = 40
CP = 8
BM = B * M
BN = B * N
NW = 32
RPW = BM // NW
GR = 8
NG = RPW // GR
GI = GR * K // 128
PW = 128
CH0 = 256
NS0 = CH0 // 16
CHF = 512
NSF = CHF // 16
NCF = N // CHF
NEG = np.float32(-3.0e38)


def _point_mlp_body(pos_ref, feat_ref, wp_ref, wf_ref, o_ref):
    p = (jnp.dot(pos_ref[...], wp_ref[...],
                 preferred_element_type=jnp.float32)
         + jnp.dot(feat_ref[...], wf_ref[...],
                   preferred_element_type=jnp.float32))
    o_ref[...] = jnp.concatenate(
        [p, jnp.zeros((p.shape[0], PW - OP_CH), jnp.float32)], axis=1)


def _head_body(q_ref, c_ref, wp_ref, bop_ref, wa_ref, ba_ref, o_ref):
    cq = jnp.dot(c_ref[...], wp_ref[...], preferred_element_type=jnp.float32)
    pooled = jnp.maximum(q_ref[:, :OP_CH] - cq + bop_ref[...], 0.0)
    o_ref[...] = jnp.maximum(
        jnp.dot(pooled, wa_ref[...], preferred_element_type=jnp.float32)
        + ba_ref[...], 0.0)


def _lane_gather(x, idx):
    return lax.gather(
        x, idx[:, None],
        dimension_numbers=lax.GatherDimensionNumbers(
            offset_dims=(), collapsed_slice_dims=(0,), start_index_map=(0,)),
        slice_sizes=(1,),
        mode=lax.GatherScatterMode.PROMISE_IN_BOUNDS)


def _sc_body(dist_hbm, p_hbm, q_hbm,
             dbuf, fbuf, ids, idsall, prows, qloc, cnts, cnt_ref,
             gsem, hsem):
    cid = lax.axis_index("c")
    sid = lax.axis_index("s")
    wid = sid * 2 + cid
    row0 = wid * RPW
    gbase = (row0 // M) * N
    iot = lax.iota(jnp.int32, 16)
    one = jnp.full((16,), 1, jnp.int32)
    zero = jnp.full((16,), 0, jnp.int32)
    negv = jnp.full((16,), NEG, jnp.float32)

    pidx = [jnp.maximum(iot - s, 0) for s in (1, 2, 4, 8)]
    pmsk = [iot >= s for s in (1, 2, 4, 8)]

    def subchunk(v, base, c):
        m = v < R2
        x = jnp.where(m, one, zero)
        for i in range(4):
            sh = _lane_gather(x, pidx[i])
            x = x + jnp.where(pmsk[i], sh, zero)
        pc = x[15]
        r = zero
        for s in (8, 4, 2, 1):
            t = r + s
            ok = _lane_gather(x, t - 1) <= iot
            r = jnp.where(ok, t, r)
        ids[pl.ds(jnp.minimum(c, 34), 16)] = r + base
        return c + pc

    def scan_row(r, grow0, par):
        cnt_ref[0] = jnp.int32(0)
        prow = par * GR + r

        def head(jq, carry):
            @pl.when(cnt_ref[0] < K)
            def _():
                c = cnt_ref[0]
                for j4 in range(4):
                    j = jq * 4 + j4
                    c = subchunk(dbuf[prow, pl.ds(j * 16, 16)],
                                 j * 16 + gbase, c)
                cnt_ref[0] = c
            return carry

        lax.fori_loop(0, NS0 // 4, head, 0)

        @pl.when(cnt_ref[0] < K)
        def _():
            def fb_chunk(f, carry):
                @pl.when(cnt_ref[0] < K)
                def _():
                    pltpu.sync_copy(
                        dist_hbm.at[pl.ds(grow0, GR), pl.ds(f * CHF, CHF)],
                        fbuf)

                    def fb_sub(j, c2):
                        @pl.when(cnt_ref[0] < K)
                        def _():
                            cnt_ref[0] = subchunk(
                                fbuf[r, pl.ds(j * 16, 16)],
                                f * CHF + j * 16 + gbase, cnt_ref[0])
                        return c2

                    lax.fori_loop(0, NSF, fb_sub, 0)
                return carry

            lax.fori_loop(1, NCF, fb_chunk, 0)

        cnt = cnt_ref[0]
        cntc = jnp.minimum(cnt, K)
        fill = _lane_gather(ids[pl.ds(0, 16)], zero)
        ids[pl.ds(cntc, 16)] = fill
        ids[pl.ds(cntc + 16, 16)] = fill
        rq = par * GI + r // 4
        rr = (r % 4) * K
        idsall[rq, pl.ds(rr, 16)] = ids[pl.ds(0, 16)]
        idsall[rq, pl.ds(rr + 16, 16)] = ids[pl.ds(16, 16)]
        cnts[prow] = cnt

    def head_copy(g, par):
        return pltpu.make_async_copy(
            dist_hbm.at[pl.ds(row0 + g * GR, GR), pl.ds(0, CH0)],
            dbuf.at[pl.ds(par * GR, GR)], hsem)

    def gather_copy(par, gi):
        return pltpu.make_async_copy(
            p_hbm.at[idsall.at[par * GI + gi]],
            prows.at[pl.ds(par * GR * K + gi * 128, 128)], gsem)

    def max_group(gq, par):
        def mrow(r, c2):
            accs = [negv] * 4
            for k in range(K):
                for t in range(4):
                    v = prows[par * GR * K + r * K + k, pl.ds(t * 16, 16)]
                    accs[t] = jnp.maximum(accs[t], v)
            valid = cnts[par * GR + r] > 0

            @pl.when(valid)
            def _():
                for t in range(4):
                    qloc[gq * GR + r, pl.ds(t * 16, 16)] = accs[t]

            @pl.when(jnp.logical_not(valid))
            def _():
                for t in range(4):
                    qloc[gq * GR + r, pl.ds(t * 16, 16)] = negv
            return c2

        lax.fori_loop(0, GR, mrow, 0)

    head_copy(0, 0).start()

    def group(g, carry):
        par = g & 1
        parp = 1 - par
        grow0 = row0 + g * GR
        head_copy(g, par).wait()

        @pl.when(g + 1 < NG)
        def _():
            head_copy(g + 1, parp).start()

        def srow(r, c2):
            scan_row(r, grow0, par)
            return c2

        lax.fori_loop(0, GR, srow, 0)

        @pl.when(g > 0)
        def _():
            for gi in range(GI):
                gather_copy(parp, gi).wait()
            max_group(g - 1, parp)

        for gi in range(GI):
            gather_copy(par, gi).start()
        return carry

    lax.fori_loop(0, NG, group, 0)
    parl = (NG - 1) & 1
    for gi in range(GI):
        gather_copy(parl, gi).wait()
    max_group(NG - 1, parl)
    pltpu.sync_copy(qloc, q_hbm.at[pl.ds(row0, RPW)])


_sc_ball = functools.partial(
    pl.kernel,
    out_type=jax.ShapeDtypeStruct((BM, PW), jnp.float32),
    mesh=plsc.VectorSubcoreMesh(core_axis_name="c", subcore_axis_name="s"),
    compiler_params=pltpu.CompilerParams(use_tc_tiling_on_sc=True),
    scratch_types=[
        pltpu.VMEM((2 * GR, CH0), jnp.float32),
        pltpu.VMEM((GR, CHF), jnp.float32),
        pltpu.VMEM((64,), jnp.int32),
        pltpu.VMEM((2 * GI, 128), jnp.int32),
        pltpu.VMEM((2 * GR * K, PW), jnp.float32),
        pltpu.VMEM((RPW, PW), jnp.float32),
        pltpu.SMEM((2 * GR,), jnp.int32),
        pltpu.SMEM((1,), jnp.int32),
        pltpu.SemaphoreType.DMA,
        pltpu.SemaphoreType.DMA,
    ],
)(_sc_body)


@jax.jit
def kernel(positions, features, centers, distances, W_op, b_op, W_agg, b_agg):
    c8 = jnp.pad(centers.reshape(BM, D), ((0, 0), (0, CP - D)))
    wp8 = jnp.pad(W_op[:D], ((0, CP - D), (0, 0)))

    p = pl.pallas_call(
        _point_mlp_body,
        grid=(BN // 8192,),
        in_specs=[pl.BlockSpec((8192, D), lambda i: (i, 0)),
                  pl.BlockSpec((8192, C), lambda i: (i, 0)),
                  pl.BlockSpec((D, OP_CH), lambda i: (0, 0)),
                  pl.BlockSpec((C, OP_CH), lambda i: (0, 0))],
        out_specs=pl.BlockSpec((8192, PW), lambda i: (i, 0)),
        out_shape=jax.ShapeDtypeStruct((BN, PW), jnp.float32),
    )(positions.reshape(BN, D), features.reshape(BN, C),
      W_op[:D], W_op[D:])

    q = _sc_ball(distances.reshape(BM, N), p)

    out = pl.pallas_call(
        _head_body,
        grid=(BM // 2048,),
        in_specs=[pl.BlockSpec((2048, PW), lambda i: (i, 0)),
                  pl.BlockSpec((2048, CP), lambda i: (i, 0)),
                  pl.BlockSpec((CP, OP_CH), lambda i: (0, 0)),
                  pl.BlockSpec((1, OP_CH), lambda i: (0, 0)),
                  pl.BlockSpec((OP_CH, OUT_CH), lambda i: (0, 0)),
                  pl.BlockSpec((1, OUT_CH), lambda i: (0, 0))],
        out_specs=pl.BlockSpec((2048, OUT_CH), lambda i: (i, 0)),
        out_shape=jax.ShapeDtypeStruct((BM, OUT_CH), jnp.float32),
    )(q, c8, wp8, b_op.reshape(1, OP_CH), W_agg, b_agg.reshape(1, OUT_CH))

    return out.reshape(B, M, OUT_CH)
